# baseline (speedup 1.0000x reference)
import functools

import jax
import jax.numpy as jnp
from jax import lax
from jax.experimental import pallas as pl
from jax.experimental.pallas import tpu as pltpu
from jax.experimental.pallas import tpu_sc as plsc

BATCH = 4096
HIST = 200
DIM = 64
LANES = 16
NDREG = DIM // LANES
TBL_ROWS = 500000
TBL_D = 2 * DIM

CHUNKS = ((0, 128), (128, 72))
NFULL = HIST // LANES
TAIL = HIST - NFULL * LANES
NBUF = 2


@functools.lru_cache(maxsize=None)
def _make_kernel(num_cores, num_subcores):
    nw = num_cores * num_subcores
    bpw = BATCH // nw
    npre = bpw * HIST // LANES
    mesh = plsc.VectorSubcoreMesh(
        core_axis_name="c", subcore_axis_name="s",
        num_cores=num_cores, num_subcores=num_subcores)

    @functools.partial(
        pl.kernel,
        out_type=jax.ShapeDtypeStruct((BATCH * DIM,), jnp.float32),
        mesh=mesh,
        scratch_types=[
            pltpu.VMEM((bpw * HIST,), jnp.int32),
            pltpu.VMEM((bpw * HIST,), jnp.float32),
            pltpu.VMEM((NBUF, HIST, TBL_D), jnp.float32),
            pltpu.VMEM((bpw * DIM,), jnp.float32),
        ] + [pltpu.SemaphoreType.DMA] * NBUF,
        compiler_params=pltpu.CompilerParams(use_tc_tiling_on_sc=True),
    )
    def emb_kernel(x_hbm, w_hbm, table_hbm, out_hbm, idx_v, w_v, rows_v,
                   out_v, *sems):
        wid = lax.axis_index("s") * num_cores + lax.axis_index("c")
        base = wid * (bpw * HIST)
        pltpu.sync_copy(x_hbm.at[pl.ds(base, bpw * HIST)], idx_v)
        pltpu.sync_copy(w_hbm.at[pl.ds(base, bpw * HIST)], w_v)

        def pre(i, carry):
            sl = pl.ds(i * LANES, LANES)
            iv = idx_v[sl]
            wv = w_v[sl]
            odd = (iv & 1) == 1
            w_v[sl] = jnp.where(odd, -wv, wv)
            idx_v[sl] = lax.shift_right_logical(iv, 1)
            return carry

        lax.fori_loop(0, npre, pre, 0)

        def issue(b, p):
            for off, sz in CHUNKS:
                pltpu.async_copy(
                    table_hbm.at[idx_v.at[pl.ds(b * HIST + off, sz)]],
                    rows_v.at[p, pl.ds(off, sz)], sems[p])

        def drain(b, p):
            for off, sz in CHUNKS:
                pltpu.make_async_copy(
                    table_hbm.at[idx_v.at[pl.ds(b * HIST + off, sz)]],
                    rows_v.at[p, pl.ds(off, sz)], sems[p]).wait()

        for p in range(NBUF):
            issue(p, p)

        def outer(g, carry):
            for p in range(NBUF):
                b = g * NBUF + p
                drain(b, p)

                def accumulate(gbase, ks, acc):
                    wv = w_v[pl.ds(b * HIST + gbase, LANES)]
                    for k in ks:
                        wl = wv[k]
                        half = jnp.where(wl < 0.0, DIM, 0)
                        wa = jnp.abs(wl)
                        acc = tuple(
                            acc[d] + wa * rows_v[p, gbase + k,
                                                 pl.ds(half + LANES * d,
                                                       LANES)]
                            for d in range(NDREG))
                    return acc

                def inner(gg, acc):
                    return accumulate(LANES * gg, range(LANES), acc)

                acc = lax.fori_loop(
                    0, NFULL, inner,
                    tuple(jnp.zeros((LANES,), jnp.float32)
                          for _ in range(NDREG)))
                acc = accumulate(HIST - LANES, range(LANES - TAIL, LANES),
                                 acc)
                for d in range(NDREG):
                    out_v[pl.ds(b * DIM + LANES * d, LANES)] = acc[d]

                @pl.when(b + NBUF < bpw)
                def _():
                    issue(b + NBUF, p)
            return carry

        lax.fori_loop(0, bpw // NBUF, outer, 0)
        pltpu.sync_copy(out_v, out_hbm.at[pl.ds(wid * (bpw * DIM),
                                                bpw * DIM)])

    return emb_kernel


def kernel(x, w, weight):
    try:
        info = plsc.get_sparse_core_info()
        nc, ns = info.num_cores, info.num_subcores
    except Exception:
        nc, ns = 2, 16
    out = _make_kernel(nc, ns)(
        x.astype(jnp.int32).reshape(-1), w.reshape(-1),
        weight.reshape(TBL_ROWS, TBL_D))
    return out.reshape(BATCH, DIM)
